# baseline (speedup 1.0000x reference)
import math

import jax
import jax.numpy as jnp
from jax import lax
from jax.experimental import pallas as pl
from jax.experimental.pallas import tpu as pltpu
from jax.experimental.pallas import tpu_sc as plsc

NC = 2
NS = 16
L = 16
NW = NC * NS

B = 4
T = 8192
D = 768
SCALE = math.sqrt(float(D))

TPW = T // NW
CH = 32
NTC = TPW // CH
VPR = D // L


def _emb_kernel(ids_hbm, tok_hbm, pos_hbm, out_hbm,
                idx_v, tok0, tok1, posbuf, spm,
                gs0, gs1, ps, xs0, xs1, os0, os1):
    cid = lax.axis_index("c")
    sid = lax.axis_index("s")
    wid = sid * NC + cid
    t0 = wid * TPW

    for b in range(B):
        pltpu.sync_copy(ids_hbm.at[pl.ds(b * T + t0, TPW)],
                        idx_v.at[pl.ds(b * TPW, TPW)])

    toks = (tok0, tok1)
    gsems = (gs0, gs1)
    xsems = (xs0, xs1)
    osems = (os0, os1)

    def out_rows(s_tc, s_b):
        return out_hbm.at[pl.ds(s_b * T + t0 + s_tc * CH, CH)]

    def wait_xbar(slot):
        pltpu.make_async_copy(toks[slot], spm.at[sid, slot],
                              xsems[slot]).wait()

    def drain_out(spm_slot):
        pltpu.make_async_copy(spm.at[sid, spm_slot],
                              out_hbm.at[pl.ds(0, CH)],
                              osems[spm_slot]).wait()

    pltpu.async_copy(pos_hbm.at[pl.ds(t0, CH)], posbuf, ps)
    pltpu.async_copy(tok_hbm.at[idx_v.at[pl.ds(0, CH)]], tok0, gs0)

    @pl.loop(0, NTC)
    def _tc(tc):
        for b in range(B):
            p = b % 2
            q = 1 - p

            def ship_prev():
                wait_xbar(q)
                pltpu.async_copy(
                    spm.at[sid, q],
                    out_rows(tc - 1, B - 1) if b == 0
                    else out_rows(tc, b - 1),
                    osems[q])

            if b == 0:
                @pl.when(tc > 0)
                def _():
                    ship_prev()
            else:
                ship_prev()

            if b < B - 1:
                pltpu.async_copy(
                    tok_hbm.at[idx_v.at[pl.ds((b + 1) * TPW + tc * CH,
                                              CH)]],
                    toks[q], gsems[q])
            else:
                @pl.when(tc < NTC - 1)
                def _():
                    pltpu.async_copy(
                        tok_hbm.at[idx_v.at[pl.ds((tc + 1) * CH, CH)]],
                        toks[q], gsems[q])

            if b >= 2:
                drain_out(p)
            else:
                @pl.when(tc > 0)
                def _():
                    drain_out(p)

            pltpu.make_async_copy(
                tok_hbm.at[pl.ds(0, CH)], toks[p], gsems[p]).wait()
            if b == 0:
                pltpu.make_async_copy(
                    pos_hbm.at[pl.ds(0, CH)], posbuf, ps).wait()

            tbuf = toks[p]

            @pl.loop(0, CH)
            def _row(r):
                for k in range(VPR):
                    sl = pl.ds(k * L, L)
                    tbuf[r, sl] = tbuf[r, sl] * SCALE + posbuf[r, sl]

            pltpu.async_copy(tbuf, spm.at[sid, p], xsems[p])

            if b == B - 1:
                @pl.when(tc < NTC - 1)
                def _():
                    pltpu.async_copy(
                        pos_hbm.at[pl.ds(t0 + (tc + 1) * CH, CH)],
                        posbuf, ps)

    wait_xbar(1)
    drain_out(0)
    pltpu.async_copy(spm.at[sid, 1], out_rows(NTC - 1, 3), osems[1])
    drain_out(1)


@jax.jit
def _emb_call(ids_flat, token_embed, pos_embed):
    mesh = plsc.VectorSubcoreMesh(core_axis_name="c", subcore_axis_name="s")
    fn = pl.kernel(
        _emb_kernel,
        out_type=jax.ShapeDtypeStruct((B * T, D), jnp.float32),
        mesh=mesh,
        scratch_types=[
            pltpu.VMEM((B * TPW,), jnp.int32),
            pltpu.VMEM((CH, D), jnp.float32),
            pltpu.VMEM((CH, D), jnp.float32),
            pltpu.VMEM((CH, D), jnp.float32),
            pltpu.VMEM_SHARED((NS, 2, CH, D), jnp.float32),
            pltpu.SemaphoreType.DMA,
            pltpu.SemaphoreType.DMA,
            pltpu.SemaphoreType.DMA,
            pltpu.SemaphoreType.DMA,
            pltpu.SemaphoreType.DMA,
            pltpu.SemaphoreType.DMA,
            pltpu.SemaphoreType.DMA,
        ],
    )
    return fn(ids_flat, token_embed, pos_embed)


def kernel(input_ids, token_embed, pos_embed):
    ids_flat = input_ids.astype(jnp.int32).reshape(B * T)
    out = _emb_call(ids_flat, token_embed, pos_embed)
    return out.reshape(B, T, D)
